# baseline (speedup 1.0000x reference)
import jax
import jax.numpy as jnp
from jax.experimental import pallas as pl
from jax.experimental.pallas import tpu as pltpu

HID = 1024
NE = 16
FF = 512
SFF = 2048
T = 512
EPS = 2
STEPS = NE // EPS
TILE = 128

_PREC = jax.lax.Precision.DEFAULT


def _dot_t(a, b, precision=_PREC):
    return jax.lax.dot_general(
        a, b, (((1,), (1,)), ((), ())),
        preferred_element_type=jnp.float32,
        precision=precision)


def _moe_body(x_ref, gate_w_ref, segw_ref, egu_ref, edw_ref, sg_ref, su_ref,
              sdw_ref, out_ref, combine_ref, rank_ref, xb_ref, sharedw_ref):
    step = pl.program_id(0)
    x = x_ref[...]

    @pl.when(step == 0)
    def _init():
        logits = _dot_t(x, gate_w_ref[...])
        idx = jax.lax.broadcasted_iota(jnp.int32, (T, NE), 1)
        m1 = jnp.max(logits, axis=1, keepdims=True)
        i1 = jnp.min(jnp.where(logits == m1, idx, NE), axis=1, keepdims=True)
        masked = jnp.where(idx == i1, -jnp.inf, logits)
        m2 = jnp.max(masked, axis=1, keepdims=True)
        i2 = jnp.min(jnp.where(masked == m2, idx, NE), axis=1, keepdims=True)
        w1 = 1.0 / (1.0 + jnp.exp(m2 - m1))
        w2 = 1.0 - w1
        combine = (jnp.where(idx == i1, w1, 0.0)
                   + jnp.where(idx == i2, w2, 0.0))
        combine_ref[...] = combine
        mask = (combine > 0.0).astype(jnp.float32)
        ltri = (jax.lax.broadcasted_iota(jnp.int32, (T, T), 1)
                < jax.lax.broadcasted_iota(jnp.int32, (T, T), 0)
                ).astype(jnp.float32)
        rank_ref[...] = jnp.dot(ltri, mask, preferred_element_type=jnp.float32,
                                precision=_PREC).astype(jnp.int32)
        xb_ref[...] = x.astype(jnp.bfloat16)
        sw = _dot_t(x, segw_ref[...])
        sharedw_ref[...] = jax.nn.sigmoid(sw)
        out_ref[...] = jnp.zeros_like(out_ref)

    xb = xb_ref[...]
    slot_iota = jax.lax.broadcasted_iota(jnp.int32, (T, TILE), 1)

    def _expert_tile(j, tt, c_col, rank_col, mask_col):
        slot = rank_col - tt * TILE
        hit = (slot == slot_iota) & mask_col
        pt = jnp.where(hit, 1.0, 0.0).astype(jnp.bfloat16)
        xt = jax.lax.dot_general(pt, xb, (((0,), (0,)), ((), ())),
                                 preferred_element_type=jnp.float32,
                                 precision=_PREC)
        gu = _dot_t(xt, egu_ref[j])
        gate, up = gu[:, :FF], gu[:, FF:]
        act = gate * jax.nn.sigmoid(gate) * up
        eout = _dot_t(act, edw_ref[j])
        ptc = jnp.where(hit, c_col, 0.0).astype(jnp.bfloat16)
        return jax.lax.dot_general(
            ptc, eout.astype(jnp.bfloat16), (((1,), (0,)), ((), ())),
            preferred_element_type=jnp.float32, precision=_PREC)

    def _expert(j):
        e = step * EPS + j
        onehot_f = (jax.lax.broadcasted_iota(jnp.int32, (1, NE), 1) == e
                    ).astype(jnp.float32)
        c_col = jnp.sum(combine_ref[...] * onehot_f, axis=1,
                        keepdims=True)
        rank_col = jnp.sum(rank_ref[...] * onehot_f.astype(jnp.int32),
                           axis=1, keepdims=True)
        mask_col = c_col > 0.0
        contrib = _expert_tile(j, 0, c_col, rank_col, mask_col)

        count = jnp.sum(mask_col.astype(jnp.int32))
        n_tiles = (count + (TILE - 1)) // TILE

        def _body(tt, carry):
            out_ref[...] += _expert_tile(j, tt, c_col, rank_col, mask_col)
            return carry
        jax.lax.fori_loop(1, n_tiles, _body, 0)

        return contrib

    g = _dot_t(xb, sg_ref[...].astype(jnp.bfloat16))
    u = _dot_t(xb, su_ref[...].astype(jnp.bfloat16))
    a = g * jax.nn.sigmoid(g) * u
    sout = _dot_t((a * sharedw_ref[...]).astype(jnp.bfloat16),
                  sdw_ref[...].astype(jnp.bfloat16))

    out_ref[...] += sout + _expert(0) + _expert(1)


def kernel(hidden_states, gate_w, expert_gate_up_w, expert_down_w,
           shared_gate_up_w, shared_down_w, shared_expert_gate_w):
    orig_shape = hidden_states.shape
    x = hidden_states.reshape(T, HID)
    sc = SFF // STEPS

    out = pl.pallas_call(
        _moe_body,
        grid=(STEPS,),
        in_specs=[
            pl.BlockSpec((T, HID), lambda s: (0, 0)),
            pl.BlockSpec((NE, HID), lambda s: (0, 0)),
            pl.BlockSpec((1, HID), lambda s: (0, 0)),
            pl.BlockSpec((EPS, 2 * FF, HID), lambda s: (s, 0, 0)),
            pl.BlockSpec((EPS, HID, FF), lambda s: (s, 0, 0)),
            pl.BlockSpec((sc, HID), lambda s: (s, 0)),
            pl.BlockSpec((sc, HID), lambda s: (s + STEPS, 0)),
            pl.BlockSpec((HID, sc), lambda s: (0, s)),
        ],
        out_specs=pl.BlockSpec((T, HID), lambda s: (0, 0)),
        out_shape=jax.ShapeDtypeStruct((T, HID), jnp.float32),
        scratch_shapes=[
            pltpu.VMEM((T, NE), jnp.float32),
            pltpu.VMEM((T, NE), jnp.int32),
            pltpu.VMEM((T, HID), jnp.bfloat16),
            pltpu.VMEM((T, 1), jnp.float32),
        ],
        compiler_params=pltpu.CompilerParams(
            dimension_semantics=("arbitrary",)),
    )(x, gate_w, shared_expert_gate_w, expert_gate_up_w, expert_down_w,
      shared_gate_up_w, shared_gate_up_w, shared_down_w)
    return out.reshape(orig_shape)
